# Initial kernel scaffold; baseline (speedup 1.0000x reference)
import functools

import jax
import jax.numpy as jnp
from jax import lax
from jax.experimental import pallas as pl
from jax.experimental.pallas import tpu as pltpu
from jax.experimental.pallas import tpu_sc as plsc

VOCAB = 1000000
EMBED_DIM = 32
BATCH = 16384
FIELDS = 26

NC = 2
NS = 16
LANES = 16
NW = NC * NS

B = BATCH * FIELDS
B_PER_W = B // NW
CHUNK = 1664
N_CHUNKS = B_PER_W // CHUNK
GROUPS = CHUNK // LANES


def _body(table_hbm, idx_hbm, out_hbm, idx_raw, idx_safe, rows, sem):
    wid = lax.axis_index("s") * NC + lax.axis_index("c")
    base = wid * B_PER_W

    @pl.loop(0, N_CHUNKS)
    def _chunk(k):
        off = base + k * CHUNK
        pltpu.sync_copy(idx_hbm.at[pl.ds(off, CHUNK)], idx_raw)

        def _grp(g, zc):
            v = idx_raw[pl.ds(g * LANES, LANES)]
            idx_safe[pl.ds(g * LANES, LANES)] = jnp.maximum(v, 1) - 1
            return zc + jnp.max(jnp.where(v == 0, 1, 0))

        zc = lax.fori_loop(0, GROUPS, _grp, jnp.int32(0))

        pltpu.async_copy(table_hbm.at[idx_safe], rows, sem).wait()

        @pl.when(zc > 0)
        def _fixup():
            zeros = jnp.zeros((LANES,), jnp.float32)

            def _fix(g, c):
                v = idx_raw[pl.ds(g * LANES, LANES)]
                m = v == 0

                @pl.when(jnp.max(jnp.where(m, 1, 0)) > 0)
                def _():
                    row_ids = g * LANES + lax.iota(jnp.int32, 16)
                    for col in range(EMBED_DIM):
                        plsc.store_scatter(
                            rows,
                            [row_ids, jnp.full((LANES,), col, jnp.int32)],
                            zeros,
                            mask=m,
                        )

                return c

            lax.fori_loop(0, GROUPS, _fix, jnp.int32(0))

        pltpu.sync_copy(rows, out_hbm.at[pl.ds(off, CHUNK)])


@functools.partial(
    pl.kernel,
    out_type=jax.ShapeDtypeStruct((B, EMBED_DIM), jnp.float32),
    mesh=plsc.VectorSubcoreMesh(core_axis_name="c", subcore_axis_name="s"),
    scratch_types=[
        pltpu.VMEM((CHUNK,), jnp.int32),
        pltpu.VMEM((CHUNK,), jnp.int32),
        pltpu.VMEM((CHUNK, EMBED_DIM), jnp.float32),
        pltpu.SemaphoreType.DMA,
    ],
)
def _sc_embed(table_hbm, idx_hbm, out_hbm, idx_raw, idx_safe, rows, sem):
    _body(table_hbm, idx_hbm, out_hbm, idx_raw, idx_safe, rows, sem)


def kernel(inputs, embedding_matrix):
    idx = inputs.reshape(-1).astype(jnp.int32)
    out = _sc_embed(embedding_matrix, idx)
    return out.reshape(inputs.shape[0], inputs.shape[1], EMBED_DIM)


# speedup vs baseline: 1.4148x; 1.4148x over previous
import functools

import jax
import jax.numpy as jnp
from jax import lax
from jax.experimental import pallas as pl
from jax.experimental.pallas import tpu as pltpu
from jax.experimental.pallas import tpu_sc as plsc

VOCAB = 1000000
EMBED_DIM = 32
BATCH = 16384
FIELDS = 26

NC = 2
NS = 16
LANES = 16
NW = NC * NS

B = BATCH * FIELDS
B_PER_W = BATCH // NW
BCHUNK = 64
N_CHUNKS = B_PER_W // BCHUNK
CHUNK = BCHUNK * FIELDS
GROUPS = CHUNK // LANES


def _body(table_hbm, idx_hbm, out_hbm, idx_raw, idx_safe, rows, oblk, sem):
    wid = lax.axis_index("s") * NC + lax.axis_index("c")
    bbase = wid * B_PER_W

    iota = lax.iota(jnp.int32, 16)
    ridx0 = iota * FIELDS

    @pl.loop(0, N_CHUNKS)
    def _chunk(k):
        b0 = bbase + k * BCHUNK
        off = b0 * FIELDS
        pltpu.sync_copy(idx_hbm.at[pl.ds(off, CHUNK)], idx_raw)

        def _grp(g, minv):
            v = idx_raw[pl.ds(g * LANES, LANES)]
            idx_safe[pl.ds(g * LANES, LANES)] = jnp.maximum(v, 1) - 1
            return jnp.minimum(minv, v)

        minv = lax.fori_loop(0, GROUPS, _grp, jnp.full((LANES,), VOCAB, jnp.int32))

        pltpu.async_copy(table_hbm.at[idx_safe], rows, sem).wait()

        chunk_min = minv[0]
        for lane in range(1, LANES):
            chunk_min = jnp.minimum(chunk_min, minv[lane])

        @pl.when(chunk_min == 0)
        def _fixup():
            zeros = jnp.zeros((LANES,), jnp.float32)

            def _fix(g, c):
                v = idx_raw[pl.ds(g * LANES, LANES)]
                m = v == 0
                row_ids = g * LANES + iota
                for col in range(EMBED_DIM):
                    plsc.store_scatter(
                        rows,
                        [row_ids, jnp.full((LANES,), col, jnp.int32)],
                        zeros,
                        mask=m,
                    )
                return c

            lax.fori_loop(0, GROUPS, _fix, jnp.int32(0))

        @pl.loop(0, FIELDS)
        def _shuf(f):
            for kk in range(BCHUNK // LANES):
                ridx = ridx0 + (kk * LANES * FIELDS + f)
                for d in range(EMBED_DIM):
                    v = plsc.load_gather(
                        rows, [ridx, jnp.full((LANES,), d, jnp.int32)]
                    )
                    oblk[f, d, pl.ds(kk * LANES, LANES)] = v

        pltpu.sync_copy(oblk, out_hbm.at[:, :, pl.ds(b0, BCHUNK)])


@functools.cache
def _sc_embed():
    return pl.kernel(
        _body,
        out_type=jax.ShapeDtypeStruct((FIELDS, EMBED_DIM, BATCH), jnp.float32),
        mesh=plsc.VectorSubcoreMesh(
            core_axis_name="c", subcore_axis_name="s", num_cores=NC, num_subcores=NS
        ),
        scratch_types=[
            pltpu.VMEM((CHUNK,), jnp.int32),
            pltpu.VMEM((CHUNK,), jnp.int32),
            pltpu.VMEM((CHUNK, EMBED_DIM), jnp.float32),
            pltpu.VMEM((FIELDS, EMBED_DIM, BCHUNK), jnp.float32),
            pltpu.SemaphoreType.DMA,
        ],
        compiler_params=pltpu.CompilerParams(
            needs_layout_passes=False, use_tc_tiling_on_sc=False
        ),
    )


def kernel(inputs, embedding_matrix):
    idx = inputs.reshape(-1).astype(jnp.int32)
    out_t = _sc_embed()(embedding_matrix, idx)
    return out_t.transpose(2, 0, 1)
